# baseline (speedup 1.0000x reference)
import functools

import jax
import jax.numpy as jnp
from jax import lax
from jax.experimental import pallas as pl
from jax.experimental.pallas import tpu as pltpu
from jax.experimental.pallas import tpu_sc as plsc

V = 1000000
D = 64
NW = 32
NPW = 3328
DUMP_W = 3456
FINE_SHIFT = 9
NFINE = 2048
COARSE_SHIFT = 15
WIN = 512
RES = 256
TAILV = V - 128
WCAP = (V - WIN) // 128 * 128
OUT_ROWS = 106624
PAD_ROW = 106560


def _splat(x):
    return jnp.full((16,), x, jnp.int32)


def _sload(ref, r, c):
    return plsc.load_gather(ref, [_splat(r), _splat(c)])[0]


def _make_route(F, B):
    n_vec = F * B // (16 * NW)
    mesh = plsc.VectorSubcoreMesh(core_axis_name="c", subcore_axis_name="s")

    @functools.partial(
        pl.kernel,
        mesh=mesh,
        out_type=(
            jax.ShapeDtypeStruct((NW, 2, DUMP_W), jnp.int32),
            jax.ShapeDtypeStruct((NW * NFINE, ), jnp.int32),
        ),
        compiler_params=pltpu.CompilerParams(needs_layout_passes=False),
        scratch_types=[
            pltpu.VMEM((F, 128), jnp.int32),
            pltpu.VMEM((NPW + 32, ), jnp.int32),
            pltpu.VMEM((NPW + 32, ), jnp.int32),
            pltpu.VMEM((DUMP_W, ), jnp.int32),
            pltpu.VMEM((DUMP_W, ), jnp.int32),
            pltpu.VMEM((NFINE, ), jnp.int32),
        ],
    )
    def route(idx_hbm, dump_vi, starts_hbm, myidx, tmpv, tmpi, pv, pi,
              stv):
        wid = lax.axis_index("s") * 2 + lax.axis_index("c")
        pltpu.sync_copy(
            idx_hbm.at[:, pl.ds(pl.multiple_of(wid * 128, 128), 128)], myidx)
        iota = lax.iota(jnp.int32, 16)
        iota_f = iota * F
        ibase = wid * 128 * F

        def coarse_body(cb, cur):
            def scan_a(t, tcur):
                f = t >> 3
                k = t & 7
                v = myidx[f, pl.ds(k * 16, 16)]
                m = (v >> COARSE_SHIFT) == cb
                ivec = iota_f + (ibase + k * 16 * F + f)
                plsc.store_compressed(tmpv.at[pl.ds(tcur, 16)], v, mask=m)
                plsc.store_compressed(tmpi.at[pl.ds(tcur, 16)], ivec, mask=m)
                return tcur + plsc.all_reduce_population_count(m)[0]

            cnt = lax.fori_loop(0, n_vec, scan_a, 0)
            trips = (cnt + 15) >> 4

            def fine_body(fbl, cur2):
                fb = cb * 64 + fbl
                plsc.store_scatter(stv, [_splat(fb)], _splat(cur2),
                                   mask=iota == 0)

                def scan_b(t, cur3):
                    v = tmpv[pl.ds(t * 16, 16)]
                    ii = tmpi[pl.ds(t * 16, 16)]
                    pos = _splat(t * 16) + iota
                    m = ((v >> FINE_SHIFT) == fb) & (pos < cnt)
                    plsc.store_compressed(pv.at[pl.ds(cur3, 16)], v, mask=m)
                    plsc.store_compressed(pi.at[pl.ds(cur3, 16)], ii, mask=m)
                    return cur3 + plsc.all_reduce_population_count(m)[0]

                return lax.fori_loop(0, trips, scan_b, cur2)

            return lax.fori_loop(0, 64, fine_body, cur)

        lax.fori_loop(0, 32, coarse_body, 0)
        pltpu.sync_copy(pv, dump_vi.at[wid, 0])
        pltpu.sync_copy(pi, dump_vi.at[wid, 1])
        pltpu.sync_copy(
            stv,
            starts_hbm.at[pl.ds(pl.multiple_of(wid * NFINE, 128), NFINE)])

    return route


def _make_gather():
    mesh = plsc.VectorSubcoreMesh(core_axis_name="c", subcore_axis_name="s")

    @functools.partial(
        pl.kernel,
        mesh=mesh,
        out_type=jax.ShapeDtypeStruct((OUT_ROWS, 128), jnp.float32),
        compiler_params=pltpu.CompilerParams(needs_layout_passes=False),
        scratch_types=[
            pltpu.VMEM((D, WIN + 128), jnp.float32),
            pltpu.VMEM((2 * NW, RES), jnp.int32),
            pltpu.VMEM((NFINE, ), jnp.int32),
            pltpu.VMEM((NW, 80), jnp.int32),
            pltpu.VMEM((48, ), jnp.int32),
            pltpu.VMEM((48, ), jnp.int32),
            pltpu.VMEM((128, 128), jnp.float32),
            pltpu.VMEM((128, ), jnp.int32),
            pltpu.SMEM((4, ), jnp.int32),
            pltpu.SemaphoreType.DMA,
        ],
    )
    def gather(table_hbm, tail_hbm, dump_vi, starts_hbm, out_hbm,
               stage, resvi, st_row, stmy, selv, seli, rowbuf, rowidx,
               smem, sem):
        wid = lax.axis_index("s") * 2 + lax.axis_index("c")
        lo = wid << COARSE_SHIFT
        iota = lax.iota(jnp.int32, 16)

        pltpu.sync_copy(tail_hbm, stage.at[:, pl.ds(WIN, 128)])

        base = wid * 64

        def load_starts(src, _):
            pltpu.sync_copy(
                starts_hbm.at[pl.ds(pl.multiple_of(src * NFINE, 128), NFINE)],
                st_row)
            stmy[src, pl.ds(0, 16)] = st_row[pl.ds(base, 16)]
            stmy[src, pl.ds(16, 16)] = st_row[pl.ds(base + 16, 16)]
            stmy[src, pl.ds(32, 16)] = st_row[pl.ds(base + 32, 16)]
            stmy[src, pl.ds(48, 16)] = st_row[pl.ds(base + 48, 16)]
            end_at = jnp.minimum(base + 64, NFINE - 16)
            stmy[src, pl.ds(64, 16)] = st_row[pl.ds(end_at, 16)]
            return 0

        lax.fori_loop(0, NW, load_starts, 0)

        cs_a = plsc.load_gather(stmy, [iota, _splat(0)])
        cs_b = plsc.load_gather(stmy, [iota + 16, _splat(0)])
        ce_a = plsc.load_gather(stmy, [iota, _splat(64)])
        ce_b = plsc.load_gather(stmy, [iota + 16, _splat(64)])
        r_a = (ce_a - (cs_a & -128) + (RES - 1)) >> 8
        r_b = (ce_b - (cs_b & -128) + (RES - 1)) >> 8
        rounds = jnp.maximum(lax.reduce_max(r_a, (0, )),
                             lax.reduce_max(r_b, (0, )))

        def reset_rowidx():
            for g in range(8):
                rowidx[pl.ds(g * 16, 16)] = _splat(PAD_ROW)

        reset_rowidx()
        smem[0] = 0
        smem[1] = 0

        def process(cnt):
            m = iota < _splat(cnt)
            vloc = selv[pl.ds(0, 16)]
            ival = seli[pl.ds(0, 16)]
            slot = smem[0]
            plsc.store_scatter(rowidx, [_splat(slot) + iota], ival, mask=m)

            def dloop(d, _):
                vals = plsc.load_gather(stage, [_splat(d), vloc], mask=m)
                plsc.store_scatter(rowbuf, [_splat(slot) + iota, _splat(d)],
                                   vals, mask=m)
                return 0

            lax.fori_loop(0, D, dloop, 0)
            smem[0] = slot + 16

            @pl.when(slot + 16 >= 128)
            def _():
                pltpu.async_copy(rowbuf, out_hbm.at[rowidx], sem).wait()
                reset_rowidx()
                smem[0] = 0

        def round_body(r, _):
            def load_res(src, _):
                cs = _sload(stmy, src, 0)
                rba = jnp.minimum((cs & -128) + r * RES, DUMP_W - RES)
                pltpu.sync_copy(
                    dump_vi.at[src, :, pl.ds(pl.multiple_of(rba, 128), RES)],
                    resvi.at[pl.ds(src * 2, 2)])
                return 0

            lax.fori_loop(0, NW, load_res, 0)

            def sub_body(s, _):
                wstart = jnp.minimum(lo + s * WIN, WCAP)
                pltpu.sync_copy(
                    table_hbm.at[:, pl.ds(pl.multiple_of(wstart, 128), WIN)],
                    stage.at[:, pl.ds(0, WIN)])

                def src_body(src, _):
                    fs = _sload(stmy, src, s)
                    fe = _sload(stmy, src, s + 1)
                    cs = _sload(stmy, src, 0)
                    rba = jnp.minimum((cs & -128) + r * RES, DUMP_W - RES)
                    wlo = jnp.maximum(fs, rba)
                    whi = jnp.minimum(fe, rba + RES)
                    j0 = jnp.maximum(wlo - rba, 0) >> 4
                    j1 = jnp.minimum((whi - rba + 15) >> 4, RES // 16)
                    trips = jnp.maximum(j1 - j0, 0)

                    def scan(t, _):
                        j = j0 + t
                        v16 = resvi[src * 2, pl.ds(j * 16, 16)]
                        i16 = resvi[src * 2 + 1, pl.ds(j * 16, 16)]
                        pos = _splat(rba + j * 16) + iota
                        m = (pos >= wlo) & (pos < whi)
                        vloc = jnp.where(v16 < TAILV, v16 - wstart,
                                         v16 - (TAILV - WIN))
                        selcur = smem[1]
                        plsc.store_compressed(selv.at[pl.ds(selcur, 16)],
                                              vloc, mask=m)
                        plsc.store_compressed(seli.at[pl.ds(selcur, 16)],
                                              i16, mask=m)
                        newcur = selcur + \
                            plsc.all_reduce_population_count(m)[0]
                        smem[1] = newcur

                        @pl.when(newcur >= 16)
                        def _():
                            process(16)
                            selv[pl.ds(0, 16)] = selv[pl.ds(16, 16)]
                            seli[pl.ds(0, 16)] = seli[pl.ds(16, 16)]
                            smem[1] = newcur - 16

                        return 0

                    lax.fori_loop(0, trips, scan, 0)
                    return 0

                lax.fori_loop(0, NW, src_body, 0)

                selcur = smem[1]

                @pl.when(selcur > 0)
                def _():
                    process(selcur)
                    smem[1] = 0

                return 0

            lax.fori_loop(0, NFINE // NW, sub_body, 0)
            return 0

        lax.fori_loop(0, rounds, round_body, 0)

        @pl.when(smem[0] > 0)
        def _():
            pltpu.async_copy(rowbuf, out_hbm.at[rowidx], sem).wait()

    return gather


def kernel(indices, table, dummy):
    B, F = indices.shape
    N = B * F
    idx_t = indices.T
    table_t = table.T
    tail = lax.slice(table_t, (0, V - 128), (D, V))
    dump_vi, starts = _make_route(F, B)(idx_t)
    out_pad = _make_gather()(table_t, tail, dump_vi, starts)
    return out_pad[:N, :D].reshape(B, F, D)


# speedup vs baseline: 1.0435x; 1.0435x over previous
import functools

import jax
import jax.numpy as jnp
from jax import lax
from jax.experimental import pallas as pl
from jax.experimental.pallas import tpu as pltpu
from jax.experimental.pallas import tpu_sc as plsc

V = 1000000
D = 64
NW = 32
NPW = 3328
DUMP_W = 3456
FINE_SHIFT = 9
NFINE = 2048
COARSE_SHIFT = 15
WIN = 512
RES = 512
TAILV = V - 128
WCAP = (V - WIN) // 128 * 128
OUT_ROWS = 106624
PAD_ROW = 106560


def _splat(x):
    return jnp.full((16,), x, jnp.int32)


def _sload(ref, r, c):
    return plsc.load_gather(ref, [_splat(r), _splat(c)])[0]


def _make_route(F, B):
    n_vec = F * B // (16 * NW)
    mesh = plsc.VectorSubcoreMesh(core_axis_name="c", subcore_axis_name="s")

    @functools.partial(
        pl.kernel,
        mesh=mesh,
        out_type=(
            jax.ShapeDtypeStruct((NW, 2, DUMP_W), jnp.int32),
            jax.ShapeDtypeStruct((NW * NFINE, ), jnp.int32),
        ),
        compiler_params=pltpu.CompilerParams(needs_layout_passes=False),
        scratch_types=[
            pltpu.VMEM((F, 128), jnp.int32),
            pltpu.VMEM((NPW + 32, ), jnp.int32),
            pltpu.VMEM((NPW + 32, ), jnp.int32),
            pltpu.VMEM((DUMP_W, ), jnp.int32),
            pltpu.VMEM((DUMP_W, ), jnp.int32),
            pltpu.VMEM((NFINE, ), jnp.int32),
        ],
    )
    def route(idx_hbm, dump_vi, starts_hbm, myidx, tmpv, tmpi, pv, pi,
              stv):
        wid = lax.axis_index("s") * 2 + lax.axis_index("c")
        pltpu.sync_copy(
            idx_hbm.at[:, pl.ds(pl.multiple_of(wid * 128, 128), 128)], myidx)
        iota = lax.iota(jnp.int32, 16)
        iota_f = iota * F
        ibase = wid * 128 * F

        def coarse_body(cb, cur):
            def scan_a(t, tcur):
                f = t >> 3
                k = t & 7
                v = myidx[f, pl.ds(k * 16, 16)]
                m = (v >> COARSE_SHIFT) == cb
                ivec = iota_f + (ibase + k * 16 * F + f)
                plsc.store_compressed(tmpv.at[pl.ds(tcur, 16)], v, mask=m)
                plsc.store_compressed(tmpi.at[pl.ds(tcur, 16)], ivec, mask=m)
                return tcur + plsc.all_reduce_population_count(m)[0]

            cnt = lax.fori_loop(0, n_vec, scan_a, 0)
            trips = (cnt + 15) >> 4

            def fine_body(fbl, cur2):
                fb = cb * 64 + fbl
                plsc.store_scatter(stv, [_splat(fb)], _splat(cur2),
                                   mask=iota == 0)

                def scan_b(t, cur3):
                    v = tmpv[pl.ds(t * 16, 16)]
                    ii = tmpi[pl.ds(t * 16, 16)]
                    pos = _splat(t * 16) + iota
                    m = ((v >> FINE_SHIFT) == fb) & (pos < cnt)
                    plsc.store_compressed(pv.at[pl.ds(cur3, 16)], v, mask=m)
                    plsc.store_compressed(pi.at[pl.ds(cur3, 16)], ii, mask=m)
                    return cur3 + plsc.all_reduce_population_count(m)[0]

                return lax.fori_loop(0, trips, scan_b, cur2)

            return lax.fori_loop(0, 64, fine_body, cur)

        lax.fori_loop(0, 32, coarse_body, 0)
        pltpu.sync_copy(pv, dump_vi.at[wid, 0])
        pltpu.sync_copy(pi, dump_vi.at[wid, 1])
        pltpu.sync_copy(
            stv,
            starts_hbm.at[pl.ds(pl.multiple_of(wid * NFINE, 128), NFINE)])

    return route


def _make_gather():
    mesh = plsc.VectorSubcoreMesh(core_axis_name="c", subcore_axis_name="s")

    @functools.partial(
        pl.kernel,
        mesh=mesh,
        out_type=jax.ShapeDtypeStruct((OUT_ROWS, 128), jnp.float32),
        compiler_params=pltpu.CompilerParams(needs_layout_passes=False),
        scratch_types=[
            pltpu.VMEM((2 * D, WIN + 128), jnp.float32),
            pltpu.VMEM((2 * NW, RES), jnp.int32),
            pltpu.VMEM((NFINE, ), jnp.int32),
            pltpu.VMEM((NW, 80), jnp.int32),
            pltpu.VMEM((48, ), jnp.int32),
            pltpu.VMEM((48, ), jnp.int32),
            pltpu.VMEM((64, 128), jnp.float32),
            pltpu.VMEM((64, ), jnp.int32),
            pltpu.SMEM((4, ), jnp.int32),
            pltpu.SemaphoreType.DMA,
            pltpu.SemaphoreType.DMA,
            pltpu.SemaphoreType.DMA,
        ],
    )
    def gather(table_hbm, tail_hbm, dump_vi, starts_hbm, out_hbm,
               stage, resvi, st_row, stmy, selv, seli, rowbuf, rowidx,
               smem, sem, gsem0, gsem1):
        wid = lax.axis_index("s") * 2 + lax.axis_index("c")
        lo = wid << COARSE_SHIFT
        iota = lax.iota(jnp.int32, 16)

        pltpu.sync_copy(tail_hbm, stage.at[pl.ds(0, D), pl.ds(WIN, 128)])
        pltpu.sync_copy(tail_hbm, stage.at[pl.ds(D, D), pl.ds(WIN, 128)])

        base = wid * 64

        def load_starts(src, _):
            pltpu.sync_copy(
                starts_hbm.at[pl.ds(pl.multiple_of(src * NFINE, 128), NFINE)],
                st_row)
            stmy[src, pl.ds(0, 16)] = st_row[pl.ds(base, 16)]
            stmy[src, pl.ds(16, 16)] = st_row[pl.ds(base + 16, 16)]
            stmy[src, pl.ds(32, 16)] = st_row[pl.ds(base + 32, 16)]
            stmy[src, pl.ds(48, 16)] = st_row[pl.ds(base + 48, 16)]
            end_at = jnp.minimum(base + 64, NFINE - 16)
            stmy[src, pl.ds(64, 16)] = st_row[pl.ds(end_at, 16)]
            return 0

        lax.fori_loop(0, NW, load_starts, 0)

        cs_a = plsc.load_gather(stmy, [iota, _splat(0)])
        cs_b = plsc.load_gather(stmy, [iota + 16, _splat(0)])
        ce_a = plsc.load_gather(stmy, [iota, _splat(64)])
        ce_b = plsc.load_gather(stmy, [iota + 16, _splat(64)])
        r_a = (ce_a - (cs_a & -128) + (RES - 1)) >> 9
        r_b = (ce_b - (cs_b & -128) + (RES - 1)) >> 9
        rounds = jnp.maximum(lax.reduce_max(r_a, (0, )),
                             lax.reduce_max(r_b, (0, )))

        def reset_rowidx():
            for g in range(4):
                rowidx[pl.ds(g * 16, 16)] = _splat(PAD_ROW)

        reset_rowidx()
        smem[0] = 0
        smem[1] = 0

        def process(cnt, row0):
            m = iota < _splat(cnt)
            vloc = selv[pl.ds(0, 16)]
            ival = seli[pl.ds(0, 16)]
            slot = smem[0]
            plsc.store_scatter(rowidx, [_splat(slot) + iota], ival, mask=m)

            def dloop(d, _):
                vals = plsc.load_gather(stage, [_splat(row0 + d), vloc],
                                        mask=m)
                plsc.store_scatter(rowbuf, [_splat(slot) + iota, _splat(d)],
                                   vals, mask=m)
                return 0

            lax.fori_loop(0, D, dloop, 0)
            smem[0] = slot + 16

            @pl.when(slot + 16 >= 64)
            def _():
                pltpu.async_copy(rowbuf, out_hbm.at[rowidx], sem).wait()
                reset_rowidx()
                smem[0] = 0

        def issue_stage(s, r):
            wstart = jnp.minimum(lo + s * WIN, WCAP)
            src_ref = table_hbm.at[:, pl.ds(pl.multiple_of(wstart, 128), WIN)]
            bb = s & 1

            @pl.when(bb == 0)
            def _():
                pltpu.async_copy(src_ref,
                                 stage.at[pl.ds(0, D), pl.ds(0, WIN)], gsem0)

            @pl.when(bb == 1)
            def _():
                pltpu.async_copy(src_ref,
                                 stage.at[pl.ds(D, D), pl.ds(0, WIN)], gsem1)

        def wait_stage(s):
            bb = s & 1

            @pl.when(bb == 0)
            def _():
                pltpu.make_async_copy(
                    table_hbm.at[:, pl.ds(0, WIN)],
                    stage.at[pl.ds(0, D), pl.ds(0, WIN)], gsem0).wait()

            @pl.when(bb == 1)
            def _():
                pltpu.make_async_copy(
                    table_hbm.at[:, pl.ds(0, WIN)],
                    stage.at[pl.ds(D, D), pl.ds(0, WIN)], gsem1).wait()

        def round_body(r, _):
            def issue_res(src, _):
                cs = _sload(stmy, src, 0)
                rba = jnp.minimum((cs & -128) + r * RES, DUMP_W - RES)
                pltpu.async_copy(
                    dump_vi.at[src, :, pl.ds(pl.multiple_of(rba, 128), RES)],
                    resvi.at[pl.ds(src * 2, 2)], sem)
                return 0

            lax.fori_loop(0, NW, issue_res, 0)

            def drain_res(src, _):
                pltpu.make_async_copy(dump_vi.at[0, :, pl.ds(0, RES)],
                                      resvi.at[pl.ds(0, 2)], sem).wait()
                return 0

            lax.fori_loop(0, NW, drain_res, 0)
            issue_stage(0, r)

            def sub_body(s, _):
                @pl.when(s + 1 < NFINE // NW)
                def _():
                    issue_stage(s + 1, r)

                wait_stage(s)
                row0 = (s & 1) * D
                wstart = jnp.minimum(lo + s * WIN, WCAP)

                def src_body(src, _):
                    fs = _sload(stmy, src, s)
                    fe = _sload(stmy, src, s + 1)
                    cs = _sload(stmy, src, 0)
                    rba = jnp.minimum((cs & -128) + r * RES, DUMP_W - RES)
                    wlo = jnp.maximum(fs, rba)
                    whi = jnp.minimum(fe, rba + RES)
                    j0 = jnp.maximum(wlo - rba, 0) >> 4
                    j1 = jnp.minimum((whi - rba + 15) >> 4, RES // 16)
                    trips = jnp.maximum(j1 - j0, 0)

                    def scan(t, _):
                        j = j0 + t
                        v16 = resvi[src * 2, pl.ds(j * 16, 16)]
                        i16 = resvi[src * 2 + 1, pl.ds(j * 16, 16)]
                        pos = _splat(rba + j * 16) + iota
                        m = (pos >= wlo) & (pos < whi)
                        vloc = jnp.where(v16 < TAILV, v16 - wstart,
                                         v16 - (TAILV - WIN))
                        selcur = smem[1]
                        plsc.store_compressed(selv.at[pl.ds(selcur, 16)],
                                              vloc, mask=m)
                        plsc.store_compressed(seli.at[pl.ds(selcur, 16)],
                                              i16, mask=m)
                        newcur = selcur + \
                            plsc.all_reduce_population_count(m)[0]
                        smem[1] = newcur

                        @pl.when(newcur >= 16)
                        def _():
                            process(16, row0)
                            selv[pl.ds(0, 16)] = selv[pl.ds(16, 16)]
                            seli[pl.ds(0, 16)] = seli[pl.ds(16, 16)]
                            smem[1] = newcur - 16

                        return 0

                    lax.fori_loop(0, trips, scan, 0)
                    return 0

                lax.fori_loop(0, NW, src_body, 0)

                selcur = smem[1]

                @pl.when(selcur > 0)
                def _():
                    process(selcur, row0)
                    smem[1] = 0

                return 0

            lax.fori_loop(0, NFINE // NW, sub_body, 0)
            return 0

        lax.fori_loop(0, rounds, round_body, 0)

        @pl.when(smem[0] > 0)
        def _():
            pltpu.async_copy(rowbuf, out_hbm.at[rowidx], sem).wait()

    return gather


def kernel(indices, table, dummy):
    B, F = indices.shape
    N = B * F
    idx_t = indices.T
    table_t = table.T
    tail = lax.slice(table_t, (0, V - 128), (D, V))
    dump_vi, starts = _make_route(F, B)(idx_t)
    out_pad = _make_gather()(table_t, tail, dump_vi, starts)
    return out_pad[:N, :D].reshape(B, F, D)


# speedup vs baseline: 1.0776x; 1.0327x over previous
import functools

import jax
import jax.numpy as jnp
from jax import lax
from jax.experimental import pallas as pl
from jax.experimental.pallas import tpu as pltpu
from jax.experimental.pallas import tpu_sc as plsc

V = 1000000
D = 64
NW = 32
NPW = 3328
DUMP_W = 3456
FINE_SHIFT = 9
NFINE = 2048
COARSE_SHIFT = 15
WIN = 512
RES = 512
TAILV = V - 128
WCAP = (V - WIN) // 128 * 128
OUT_ROWS = 106624
PAD_ROW = 106560


def _splat(x):
    return jnp.full((16,), x, jnp.int32)


def _sload(ref, r, c):
    return plsc.load_gather(ref, [_splat(r), _splat(c)])[0]


def _make_route(F, B):
    n_vec = F * B // (16 * NW)
    mesh = plsc.VectorSubcoreMesh(core_axis_name="c", subcore_axis_name="s")

    @functools.partial(
        pl.kernel,
        mesh=mesh,
        out_type=(
            jax.ShapeDtypeStruct((NW, 2, DUMP_W), jnp.int32),
            jax.ShapeDtypeStruct((NW * NFINE, ), jnp.int32),
        ),
        compiler_params=pltpu.CompilerParams(needs_layout_passes=False),
        scratch_types=[
            pltpu.VMEM((F, 128), jnp.int32),
            pltpu.VMEM((NPW + 32, ), jnp.int32),
            pltpu.VMEM((NPW + 32, ), jnp.int32),
            pltpu.VMEM((DUMP_W, ), jnp.int32),
            pltpu.VMEM((DUMP_W, ), jnp.int32),
            pltpu.VMEM((NFINE, ), jnp.int32),
            pltpu.VMEM((4, NPW + 32), jnp.int32),
            pltpu.VMEM((4, NPW + 32), jnp.int32),
        ],
    )
    def route(idx_hbm, dump_vi, starts_hbm, myidx, tmpv, tmpi, pv, pi,
              stv, qv, qi):
        wid = lax.axis_index("s") * 2 + lax.axis_index("c")
        pltpu.sync_copy(
            idx_hbm.at[:, pl.ds(pl.multiple_of(wid * 128, 128), 128)], myidx)
        iota = lax.iota(jnp.int32, 16)
        iota_f = iota * F
        ibase = wid * 128 * F

        def quad_scan(t, carry):
            f = t >> 3
            k = t & 7
            v = myidx[f, pl.ds(k * 16, 16)]
            ivec = iota_f + (ibase + k * 16 * F + f)
            out = []
            for q in range(4):
                m = (v >> 18) == q
                plsc.store_compressed(qv.at[q, pl.ds(carry[q], 16)], v,
                                      mask=m)
                plsc.store_compressed(qi.at[q, pl.ds(carry[q], 16)], ivec,
                                      mask=m)
                out.append(carry[q] +
                           plsc.all_reduce_population_count(m)[0])
            return tuple(out)

        qc = lax.fori_loop(0, n_vec, quad_scan, (0, 0, 0, 0))

        def coarse_body(cb, cur):
            q = cb >> 3
            qcnt = jnp.where(q == 0, qc[0],
                             jnp.where(q == 1, qc[1],
                                       jnp.where(q == 2, qc[2], qc[3])))
            qtrips = (qcnt + 15) >> 4

            def scan_a(t, tcur):
                v = qv[q, pl.ds(t * 16, 16)]
                ii = qi[q, pl.ds(t * 16, 16)]
                pos = _splat(t * 16) + iota
                m = ((v >> COARSE_SHIFT) == cb) & (pos < qcnt)
                plsc.store_compressed(tmpv.at[pl.ds(tcur, 16)], v, mask=m)
                plsc.store_compressed(tmpi.at[pl.ds(tcur, 16)], ii, mask=m)
                return tcur + plsc.all_reduce_population_count(m)[0]

            cnt = lax.fori_loop(0, qtrips, scan_a, 0)
            trips = (cnt + 15) >> 4

            def fine_body(fbl, cur2):
                fb = cb * 64 + fbl
                plsc.store_scatter(stv, [_splat(fb)], _splat(cur2),
                                   mask=iota == 0)

                def scan_b(t, cur3):
                    v = tmpv[pl.ds(t * 16, 16)]
                    ii = tmpi[pl.ds(t * 16, 16)]
                    pos = _splat(t * 16) + iota
                    m = ((v >> FINE_SHIFT) == fb) & (pos < cnt)
                    plsc.store_compressed(pv.at[pl.ds(cur3, 16)], v, mask=m)
                    plsc.store_compressed(pi.at[pl.ds(cur3, 16)], ii, mask=m)
                    return cur3 + plsc.all_reduce_population_count(m)[0]

                return lax.fori_loop(0, trips, scan_b, cur2)

            return lax.fori_loop(0, 64, fine_body, cur)

        lax.fori_loop(0, 32, coarse_body, 0)
        pltpu.sync_copy(pv, dump_vi.at[wid, 0])
        pltpu.sync_copy(pi, dump_vi.at[wid, 1])
        pltpu.sync_copy(
            stv,
            starts_hbm.at[pl.ds(pl.multiple_of(wid * NFINE, 128), NFINE)])

    return route


def _make_gather():
    mesh = plsc.VectorSubcoreMesh(core_axis_name="c", subcore_axis_name="s")

    @functools.partial(
        pl.kernel,
        mesh=mesh,
        out_type=jax.ShapeDtypeStruct((OUT_ROWS, 128), jnp.float32),
        compiler_params=pltpu.CompilerParams(needs_layout_passes=False),
        scratch_types=[
            pltpu.VMEM((2 * D, WIN + 128), jnp.float32),
            pltpu.VMEM((2 * NW, RES), jnp.int32),
            pltpu.VMEM((NFINE, ), jnp.int32),
            pltpu.VMEM((NW, 80), jnp.int32),
            pltpu.VMEM((48, ), jnp.int32),
            pltpu.VMEM((48, ), jnp.int32),
            pltpu.VMEM((64, 128), jnp.float32),
            pltpu.VMEM((64, ), jnp.int32),
            pltpu.SMEM((4, ), jnp.int32),
            pltpu.SemaphoreType.DMA,
            pltpu.SemaphoreType.DMA,
            pltpu.SemaphoreType.DMA,
        ],
    )
    def gather(table_hbm, tail_hbm, dump_vi, starts_hbm, out_hbm,
               stage, resvi, st_row, stmy, selv, seli, rowbuf, rowidx,
               smem, sem, gsem0, gsem1):
        wid = lax.axis_index("s") * 2 + lax.axis_index("c")
        lo = wid << COARSE_SHIFT
        iota = lax.iota(jnp.int32, 16)

        pltpu.sync_copy(tail_hbm, stage.at[pl.ds(0, D), pl.ds(WIN, 128)])
        pltpu.sync_copy(tail_hbm, stage.at[pl.ds(D, D), pl.ds(WIN, 128)])

        base = wid * 64

        def load_starts(src, _):
            pltpu.sync_copy(
                starts_hbm.at[pl.ds(pl.multiple_of(src * NFINE, 128), NFINE)],
                st_row)
            stmy[src, pl.ds(0, 16)] = st_row[pl.ds(base, 16)]
            stmy[src, pl.ds(16, 16)] = st_row[pl.ds(base + 16, 16)]
            stmy[src, pl.ds(32, 16)] = st_row[pl.ds(base + 32, 16)]
            stmy[src, pl.ds(48, 16)] = st_row[pl.ds(base + 48, 16)]
            end_at = jnp.minimum(base + 64, NFINE - 16)
            stmy[src, pl.ds(64, 16)] = st_row[pl.ds(end_at, 16)]
            return 0

        lax.fori_loop(0, NW, load_starts, 0)

        cs_a = plsc.load_gather(stmy, [iota, _splat(0)])
        cs_b = plsc.load_gather(stmy, [iota + 16, _splat(0)])
        ce_a = plsc.load_gather(stmy, [iota, _splat(64)])
        ce_b = plsc.load_gather(stmy, [iota + 16, _splat(64)])
        r_a = (ce_a - (cs_a & -128) + (RES - 1)) >> 9
        r_b = (ce_b - (cs_b & -128) + (RES - 1)) >> 9
        rounds = jnp.maximum(lax.reduce_max(r_a, (0, )),
                             lax.reduce_max(r_b, (0, )))

        def reset_rowidx():
            for g in range(4):
                rowidx[pl.ds(g * 16, 16)] = _splat(PAD_ROW)

        reset_rowidx()
        smem[0] = 0
        smem[1] = 0

        def process(cnt, row0):
            m = iota < _splat(cnt)
            vloc = selv[pl.ds(0, 16)]
            ival = seli[pl.ds(0, 16)]
            slot = smem[0]
            plsc.store_scatter(rowidx, [_splat(slot) + iota], ival, mask=m)

            rowvec = _splat(slot) + iota
            for d in range(D):
                vals = plsc.load_gather(stage, [_splat(row0 + d), vloc],
                                        mask=m)
                plsc.store_scatter(rowbuf, [rowvec, _splat(d)],
                                   vals, mask=m)
            smem[0] = slot + 16

            @pl.when(slot + 16 >= 64)
            def _():
                pltpu.async_copy(rowbuf, out_hbm.at[rowidx], sem).wait()
                reset_rowidx()
                smem[0] = 0

        def issue_stage(s, r):
            wstart = jnp.minimum(lo + s * WIN, WCAP)
            src_ref = table_hbm.at[:, pl.ds(pl.multiple_of(wstart, 128), WIN)]
            bb = s & 1

            @pl.when(bb == 0)
            def _():
                pltpu.async_copy(src_ref,
                                 stage.at[pl.ds(0, D), pl.ds(0, WIN)], gsem0)

            @pl.when(bb == 1)
            def _():
                pltpu.async_copy(src_ref,
                                 stage.at[pl.ds(D, D), pl.ds(0, WIN)], gsem1)

        def wait_stage(s):
            bb = s & 1

            @pl.when(bb == 0)
            def _():
                pltpu.make_async_copy(
                    table_hbm.at[:, pl.ds(0, WIN)],
                    stage.at[pl.ds(0, D), pl.ds(0, WIN)], gsem0).wait()

            @pl.when(bb == 1)
            def _():
                pltpu.make_async_copy(
                    table_hbm.at[:, pl.ds(0, WIN)],
                    stage.at[pl.ds(D, D), pl.ds(0, WIN)], gsem1).wait()

        def round_body(r, _):
            def issue_res(src, _):
                cs = _sload(stmy, src, 0)
                rba = jnp.minimum((cs & -128) + r * RES, DUMP_W - RES)
                pltpu.async_copy(
                    dump_vi.at[src, :, pl.ds(pl.multiple_of(rba, 128), RES)],
                    resvi.at[pl.ds(src * 2, 2)], sem)
                return 0

            lax.fori_loop(0, NW, issue_res, 0)

            def drain_res(src, _):
                pltpu.make_async_copy(dump_vi.at[0, :, pl.ds(0, RES)],
                                      resvi.at[pl.ds(0, 2)], sem).wait()
                return 0

            lax.fori_loop(0, NW, drain_res, 0)
            issue_stage(0, r)

            def sub_body(s, _):
                @pl.when(s + 1 < NFINE // NW)
                def _():
                    issue_stage(s + 1, r)

                wait_stage(s)
                row0 = (s & 1) * D
                wstart = jnp.minimum(lo + s * WIN, WCAP)

                def src_body(src, _):
                    fs = _sload(stmy, src, s)
                    fe = _sload(stmy, src, s + 1)
                    cs = _sload(stmy, src, 0)
                    rba = jnp.minimum((cs & -128) + r * RES, DUMP_W - RES)
                    wlo = jnp.maximum(fs, rba)
                    whi = jnp.minimum(fe, rba + RES)
                    j0 = jnp.maximum(wlo - rba, 0) >> 4
                    j1 = jnp.minimum((whi - rba + 15) >> 4, RES // 16)
                    trips = jnp.maximum(j1 - j0, 0)

                    def scan(t, _):
                        j = j0 + t
                        v16 = resvi[src * 2, pl.ds(j * 16, 16)]
                        i16 = resvi[src * 2 + 1, pl.ds(j * 16, 16)]
                        pos = _splat(rba + j * 16) + iota
                        m = (pos >= wlo) & (pos < whi)
                        vloc = jnp.where(v16 < TAILV, v16 - wstart,
                                         v16 - (TAILV - WIN))
                        selcur = smem[1]
                        plsc.store_compressed(selv.at[pl.ds(selcur, 16)],
                                              vloc, mask=m)
                        plsc.store_compressed(seli.at[pl.ds(selcur, 16)],
                                              i16, mask=m)
                        newcur = selcur + \
                            plsc.all_reduce_population_count(m)[0]
                        smem[1] = newcur

                        @pl.when(newcur >= 16)
                        def _():
                            process(16, row0)
                            selv[pl.ds(0, 16)] = selv[pl.ds(16, 16)]
                            seli[pl.ds(0, 16)] = seli[pl.ds(16, 16)]
                            smem[1] = newcur - 16

                        return 0

                    lax.fori_loop(0, trips, scan, 0)
                    return 0

                lax.fori_loop(0, NW, src_body, 0)

                selcur = smem[1]

                @pl.when(selcur > 0)
                def _():
                    process(selcur, row0)
                    smem[1] = 0

                return 0

            lax.fori_loop(0, NFINE // NW, sub_body, 0)
            return 0

        lax.fori_loop(0, rounds, round_body, 0)

        @pl.when(smem[0] > 0)
        def _():
            pltpu.async_copy(rowbuf, out_hbm.at[rowidx], sem).wait()

    return gather


def kernel(indices, table, dummy):
    B, F = indices.shape
    N = B * F
    idx_t = indices.T
    table_t = table.T
    tail = lax.slice(table_t, (0, V - 128), (D, V))
    dump_vi, starts = _make_route(F, B)(idx_t)
    out_pad = _make_gather()(table_t, tail, dump_vi, starts)
    return out_pad[:N, :D].reshape(B, F, D)


# speedup vs baseline: 2.0413x; 1.8942x over previous
import functools

import jax
import jax.numpy as jnp
from jax import lax
from jax.experimental import pallas as pl
from jax.experimental.pallas import tpu as pltpu
from jax.experimental.pallas import tpu_sc as plsc


def _make_sc_gather(V, D, N):
    info = plsc.get_sparse_core_info()
    NW = info.num_cores * info.num_subcores
    assert N % NW == 0
    n_per_w = N // NW
    CH = 832
    assert n_per_w % CH == 0
    n_ch = n_per_w // CH
    mesh = plsc.VectorSubcoreMesh(core_axis_name="c", subcore_axis_name="s")

    @functools.partial(
        pl.kernel,
        mesh=mesh,
        out_type=jax.ShapeDtypeStruct((N, D), jnp.float32),
        compiler_params=pltpu.CompilerParams(use_tc_tiling_on_sc=False),
        scratch_types=[
            pltpu.VMEM((n_per_w,), jnp.int32),
            pltpu.VMEM((2, CH, D), jnp.float32),
            pltpu.SemaphoreType.DMA,
            pltpu.SemaphoreType.DMA,
            pltpu.SemaphoreType.DMA,
            pltpu.SemaphoreType.DMA,
        ],
    )
    def gather_kernel(idx_hbm, table_hbm, out_hbm, idx_v, rows_v,
                      gsem0, gsem1, osem0, osem1):
        gsem = (gsem0, gsem1)
        osem = (osem0, osem1)
        wid = lax.axis_index("s") * info.num_cores + lax.axis_index("c")
        base = wid * n_per_w
        pltpu.sync_copy(idx_hbm.at[pl.ds(base, n_per_w)], idx_v)

        def start_gather(i):
            b = i % 2
            return pltpu.async_copy(
                table_hbm.at[idx_v.at[pl.ds(i * CH, CH)]], rows_v.at[b],
                gsem[b])

        g_cur = start_gather(0)
        out_handles = [None, None]
        for i in range(n_ch):
            b = i % 2
            if i + 1 < n_ch:
                nb = (i + 1) % 2
                if out_handles[nb] is not None:
                    out_handles[nb].wait()
                g_next = start_gather(i + 1)
            g_cur.wait()
            out_handles[b] = pltpu.async_copy(
                rows_v.at[b], out_hbm.at[pl.ds(base + i * CH, CH)], osem[b])
            if i + 1 < n_ch:
                g_cur = g_next
        for h in out_handles:
            if h is not None:
                h.wait()

    return gather_kernel


def kernel(indices, table, dummy):
    B, F = indices.shape
    V, D = table.shape
    N = B * F
    idx_flat = indices.reshape(N)
    out = _make_sc_gather(V, D, N)(idx_flat, table)
    return out.reshape(B, F, D)
